# baseline (speedup 1.0000x reference)
import functools

import jax
import jax.numpy as jnp
from jax.experimental import pallas as pl

_RES = 32
_F = 256


_RPB = 32


def _pos_embed_body(row_ref, col_ref, out_ref):
    g = pl.program_id(0)
    col = col_ref[...]
    for i in range(_RPB):
        out_ref[pl.ds(i * _RES, _RES), 0:_F] = col
        out_ref[pl.ds(i * _RES, _RES), _F:2 * _F] = jnp.broadcast_to(
            row_ref[pl.ds(g * _RPB + i, 1), :], (_RES, _F))


@jax.jit
def _pos_embed(row_embed, col_embed):
    return pl.pallas_call(
        _pos_embed_body,
        grid=(_RES // _RPB,),
        in_specs=[
            pl.BlockSpec((_RES, _F), lambda g: (0, 0)),
            pl.BlockSpec((_RES, _F), lambda g: (0, 0)),
        ],
        out_specs=pl.BlockSpec((_RPB * _RES, 2 * _F), lambda g: (g, 0)),
        out_shape=jax.ShapeDtypeStruct((_RES * _RES, 2 * _F), jnp.float32),
    )(row_embed, col_embed)


def kernel(row_embed, col_embed):
    pos = _pos_embed(row_embed, col_embed)
    return pos[None, :, :]


# speedup vs baseline: 1.0396x; 1.0396x over previous
import jax
import jax.numpy as jnp
from jax.experimental import pallas as pl
from jax.experimental.pallas import tpu as pltpu

_RES = 32
_F = 256
_NCHUNK = 8
_GPC = _RES // _NCHUNK


def _pos_embed_body(row_ref, col_ref, out_hbm, scratch, sems):
    col = col_ref[...]
    rows_per_chunk = _GPC * _RES
    for c in range(_NCHUNK):
        for i in range(_GPC):
            r = c * _GPC + i
            scratch[pl.ds(r * _RES, _RES), 0:_F] = col
            scratch[pl.ds(r * _RES, _RES), _F:2 * _F] = jnp.broadcast_to(
                row_ref[pl.ds(r, 1), :], (_RES, _F))
        pltpu.make_async_copy(
            scratch.at[pl.ds(c * rows_per_chunk, rows_per_chunk)],
            out_hbm.at[pl.ds(c * rows_per_chunk, rows_per_chunk)],
            sems.at[c],
        ).start()
    for c in range(_NCHUNK):
        pltpu.make_async_copy(
            scratch.at[pl.ds(c * rows_per_chunk, rows_per_chunk)],
            out_hbm.at[pl.ds(c * rows_per_chunk, rows_per_chunk)],
            sems.at[c],
        ).wait()


@jax.jit
def _pos_embed(row_embed, col_embed):
    return pl.pallas_call(
        _pos_embed_body,
        in_specs=[
            pl.BlockSpec(memory_space=pltpu.VMEM),
            pl.BlockSpec(memory_space=pltpu.VMEM),
        ],
        out_specs=pl.BlockSpec(memory_space=pl.ANY),
        out_shape=jax.ShapeDtypeStruct((_RES * _RES, 2 * _F), jnp.float32),
        scratch_shapes=[
            pltpu.VMEM((_RES * _RES, 2 * _F), jnp.float32),
            pltpu.SemaphoreType.DMA((_NCHUNK,)),
        ],
    )(row_embed, col_embed)


def kernel(row_embed, col_embed):
    pos = _pos_embed(row_embed, col_embed)
    return pos[None, :, :]
